# baseline (speedup 1.0000x reference)
import functools

import jax
import jax.numpy as jnp
from jax import lax
from jax.experimental import pallas as pl
from jax.experimental.pallas import tpu as pltpu
from jax.experimental.pallas import tpu_sc as plsc
from jax._src.pallas import mpmd as _mpmd

M = 1_000_000
B = 262_144
NC = 2
NS = 16
NW = NC * NS
E = 3 * M
BE = 3 * B
IDXC = 128
NCHUNK = BE // IDXC
C_PER_W = NCHUNK // NW
ZCHUNK = 12288
NZCHUNK = (E + ZCHUNK - 1) // ZCHUNK


def _mesh():
    return plsc.VectorSubcoreMesh(core_axis_name="c", subcore_axis_name="s")


_PARAMS = pltpu.CompilerParams(use_tc_tiling_on_sc=False)


def _wid():
    return lax.axis_index("s") * NC + lax.axis_index("c")


def _make_zero_kernel():
    @functools.partial(
        pl.kernel,
        mesh=_mesh(),
        out_type=jax.ShapeDtypeStruct((E,), jnp.float32),
        scratch_types=[
            pltpu.VMEM((ZCHUNK,), jnp.float32),
            pltpu.SemaphoreType.DMA,
        ],
        compiler_params=_PARAMS,
    )
    def zero_kernel(zc_hbm, out_hbm, zbuf, zsem):
        wid = _wid()
        pltpu.sync_copy(zc_hbm, zbuf)
        nj = (NZCHUNK - wid + NW - 1) // NW

        def zstart(i, _):
            chunk = wid + i * NW
            e0 = jnp.where(chunk == NZCHUNK - 1, E - ZCHUNK, chunk * ZCHUNK)
            pltpu.async_copy(zbuf, out_hbm.at[pl.ds(e0, ZCHUNK)], zsem)
            return ()

        def zdrain(i, _):
            chunk = wid + i * NW
            e0 = jnp.where(chunk == NZCHUNK - 1, E - ZCHUNK, chunk * ZCHUNK)
            pltpu.make_async_copy(zbuf, out_hbm.at[pl.ds(e0, ZCHUNK)], zsem).wait()
            return ()

        lax.fori_loop(0, nj, zstart, ())
        lax.fori_loop(0, nj, zdrain, ())

    return zero_kernel


def _make_scatter_kernel():
    def scatter_body(buf_in, idx_hbm, ones_hbm, out_hbm, idx_v, ones_v, sem):
        del buf_in
        wid = _wid()
        pltpu.sync_copy(ones_hbm, ones_v)
        pltpu.sync_copy(idx_hbm.at[pl.ds(wid * C_PER_W, C_PER_W)], idx_v)

        def sstart(j, _):
            pltpu.async_copy(ones_v, out_hbm.at[idx_v.at[j]], sem)
            return ()

        def sdrain(j, _):
            pltpu.make_async_copy(ones_v, out_hbm.at[idx_v.at[j]], sem).wait()
            return ()

        lax.fori_loop(0, C_PER_W, sstart, ())
        lax.fori_loop(0, C_PER_W, sdrain, ())

    return _mpmd._mpmd_map(
        [(_mesh(), scatter_body)],
        out_types=jax.ShapeDtypeStruct((E,), jnp.float32),
        input_output_aliases={0: 0},
        scratch_types=[
            pltpu.VMEM((C_PER_W, IDXC), jnp.int32),
            pltpu.VMEM((IDXC,), jnp.float32),
            pltpu.SemaphoreType.DMA,
        ],
        compiler_params=_PARAMS,
    )


def kernel(vertices, mask):
    del vertices
    idx = mask.astype(jnp.int32)
    idx3 = (3 * idx[:, None] + jnp.arange(3, dtype=jnp.int32)[None, :])
    idx3 = idx3.reshape(NCHUNK, IDXC)
    zconst = jnp.zeros((ZCHUNK,), jnp.float32)
    ones = jnp.ones((IDXC,), jnp.float32)
    zeros = _make_zero_kernel()(zconst)
    out = _make_scatter_kernel()(zeros, idx3, ones)
    return out.reshape(1, M, 3)


# speedup vs baseline: 2.4016x; 2.4016x over previous
import functools

import jax
import jax.numpy as jnp
from jax import lax
from jax.experimental import pallas as pl
from jax.experimental.pallas import tpu as pltpu
from jax.experimental.pallas import tpu_sc as plsc

M = 1_000_000
B = 262_144
NC = 2
NS = 16
NW = NC * NS
E = 3 * M
IDXC = 128
NCHUNK = B // IDXC
C_PER_S = NCHUNK // NS
RCH = 4_800
NRCH = (M + RCH - 1) // RCH


def _mesh():
    return plsc.VectorSubcoreMesh(core_axis_name="c", subcore_axis_name="s")


_PARAMS = pltpu.CompilerParams(
    use_tc_tiling_on_sc=False, needs_layout_passes=False
)


def _make_mask_kernel():
    @functools.partial(
        pl.kernel,
        mesh=_mesh(),
        out_type=jax.ShapeDtypeStruct((E,), jnp.float32),
        scratch_types=[
            pltpu.VMEM_SHARED((M,), jnp.float32),
            pltpu.VMEM((C_PER_S, IDXC), jnp.int32),
            pltpu.VMEM((IDXC,), jnp.float32),
            pltpu.VMEM((RCH,), jnp.float32),
            pltpu.VMEM((RCH,), jnp.float32),
            pltpu.VMEM((3 * RCH,), jnp.float32),
            pltpu.SemaphoreType.DMA,
        ],
        compiler_params=_PARAMS,
    )
    def mask_kernel(idx_hbm, zc_hbm, ones_hbm, out_hbm,
                    flags_sh, idx_v, ones_v, zb, fl_v, ob_v, zsem):
        sub = lax.axis_index("s")
        wid = sub * NC + lax.axis_index("c")
        pltpu.sync_copy(zc_hbm, zb)
        pltpu.sync_copy(ones_hbm, ones_v)
        pltpu.sync_copy(idx_hbm.at[pl.ds(sub * C_PER_S, C_PER_S)], idx_v)

        nz = (NRCH - sub + NS - 1) // NS

        def zstart(i, _):
            chunk = sub + i * NS
            r0 = jnp.where(chunk == NRCH - 1, M - RCH, chunk * RCH)
            pltpu.async_copy(zb, flags_sh.at[pl.ds(r0, RCH)], zsem)
            return ()

        def zdrain(i, _):
            chunk = sub + i * NS
            r0 = jnp.where(chunk == NRCH - 1, M - RCH, chunk * RCH)
            pltpu.make_async_copy(zb, flags_sh.at[pl.ds(r0, RCH)], zsem).wait()
            return ()

        lax.fori_loop(0, nz, zstart, ())
        lax.fori_loop(0, nz, zdrain, ())
        plsc.subcore_barrier()

        def sadd(j, _):
            pltpu.sync_copy(ones_v, flags_sh.at[idx_v.at[j]], add=True)
            return ()

        lax.fori_loop(0, C_PER_S, sadd, ())
        plsc.subcore_barrier()

        tri = 3 * lax.iota(jnp.int32, 16)
        ne = (NRCH - wid + NW - 1) // NW

        def echunk(i, _):
            chunk = wid + i * NW
            r0 = jnp.where(chunk == NRCH - 1, M - RCH, chunk * RCH)
            pltpu.sync_copy(flags_sh.at[pl.ds(r0, RCH)], fl_v)

            def evec(k, _):
                f = fl_v[pl.ds(k * 16, 16)]
                fc = jnp.minimum(f, 1.0)
                base = 48 * k
                plsc.store_scatter(ob_v, [tri + base], fc)
                plsc.store_scatter(ob_v, [tri + (base + 1)], fc)
                plsc.store_scatter(ob_v, [tri + (base + 2)], fc)
                return ()

            lax.fori_loop(0, RCH // 16, evec, ())
            pltpu.sync_copy(ob_v, out_hbm.at[pl.ds(3 * r0, 3 * RCH)])
            return ()

        lax.fori_loop(0, ne, echunk, ())

    return mask_kernel


def kernel(vertices, mask):
    del vertices
    idx = mask.astype(jnp.int32).reshape(NCHUNK, IDXC)
    zconst = jnp.zeros((RCH,), jnp.float32)
    ones = jnp.ones((IDXC,), jnp.float32)
    out = _make_mask_kernel()(idx, zconst, ones)
    return out.reshape(1, M, 3)


# speedup vs baseline: 2.4229x; 1.0088x over previous
import functools

import jax
import jax.numpy as jnp
from jax import lax
from jax.experimental import pallas as pl
from jax.experimental.pallas import tpu as pltpu
from jax.experimental.pallas import tpu_sc as plsc

M = 1_000_000
B = 262_144
NC = 2
NS = 16
NW = NC * NS
E = 3 * M
IDXC = 128
NCHUNK = B // IDXC
C_PER_S = NCHUNK // NS
RCH = 9_600
NRCH = (M + RCH - 1) // RCH


def _mesh():
    return plsc.VectorSubcoreMesh(core_axis_name="c", subcore_axis_name="s")


_PARAMS = pltpu.CompilerParams(
    use_tc_tiling_on_sc=False, needs_layout_passes=False
)


def _make_mask_kernel():
    @functools.partial(
        pl.kernel,
        mesh=_mesh(),
        out_type=jax.ShapeDtypeStruct((E,), jnp.float32),
        scratch_types=[
            pltpu.VMEM_SHARED((M,), jnp.float32),
            pltpu.VMEM((C_PER_S, IDXC), jnp.int32),
            pltpu.VMEM((IDXC,), jnp.float32),
            pltpu.VMEM((RCH,), jnp.float32),
            pltpu.VMEM((RCH,), jnp.float32),
            pltpu.VMEM((3 * RCH,), jnp.float32),
            pltpu.SemaphoreType.DMA,
        ],
        compiler_params=_PARAMS,
    )
    def mask_kernel(idx_hbm, zc_hbm, ones_hbm, out_hbm,
                    flags_sh, idx_v, ones_v, zb, fl_v, ob_v, zsem):
        sub = lax.axis_index("s")
        wid = sub * NC + lax.axis_index("c")
        pltpu.sync_copy(zc_hbm, zb)
        pltpu.sync_copy(ones_hbm, ones_v)
        pltpu.sync_copy(idx_hbm.at[pl.ds(sub * C_PER_S, C_PER_S)], idx_v)

        nz = (NRCH - sub + NS - 1) // NS

        def zstart(i, _):
            chunk = sub + i * NS
            r0 = jnp.where(chunk == NRCH - 1, M - RCH, chunk * RCH)
            pltpu.async_copy(zb, flags_sh.at[pl.ds(r0, RCH)], zsem)
            return ()

        def zdrain(i, _):
            chunk = sub + i * NS
            r0 = jnp.where(chunk == NRCH - 1, M - RCH, chunk * RCH)
            pltpu.make_async_copy(zb, flags_sh.at[pl.ds(r0, RCH)], zsem).wait()
            return ()

        lax.fori_loop(0, nz, zstart, ())
        lax.fori_loop(0, nz, zdrain, ())
        plsc.subcore_barrier()

        def sstart(j, _):
            pltpu.async_copy(ones_v, flags_sh.at[idx_v.at[j]], zsem, add=True)
            return ()

        def sdrain(j, _):
            pltpu.make_async_copy(
                ones_v, flags_sh.at[idx_v.at[j]], zsem
            ).wait()
            return ()

        lax.fori_loop(0, C_PER_S, sstart, ())
        lax.fori_loop(0, C_PER_S, sdrain, ())
        plsc.subcore_barrier()

        tri = 3 * lax.iota(jnp.int32, 16)
        ne = (NRCH - wid + NW - 1) // NW

        def echunk(i, _):
            chunk = wid + i * NW
            r0 = jnp.where(chunk == NRCH - 1, M - RCH, chunk * RCH)
            pltpu.sync_copy(flags_sh.at[pl.ds(r0, RCH)], fl_v)

            def evec(k, _):
                f = fl_v[pl.ds(k * 16, 16)]
                fc = jnp.minimum(f, 1.0)
                base = 48 * k
                plsc.store_scatter(ob_v, [tri + base], fc)
                plsc.store_scatter(ob_v, [tri + (base + 1)], fc)
                plsc.store_scatter(ob_v, [tri + (base + 2)], fc)
                return ()

            lax.fori_loop(0, RCH // 16, evec, ())
            pltpu.sync_copy(ob_v, out_hbm.at[pl.ds(3 * r0, 3 * RCH)])
            return ()

        lax.fori_loop(0, ne, echunk, ())

    return mask_kernel


def kernel(vertices, mask):
    del vertices
    idx = mask.astype(jnp.int32).reshape(NCHUNK, IDXC)
    zconst = jnp.zeros((RCH,), jnp.float32)
    ones = jnp.ones((IDXC,), jnp.float32)
    out = _make_mask_kernel()(idx, zconst, ones)
    return out.reshape(1, M, 3)
